# Initial kernel scaffold; baseline (speedup 1.0000x reference)
import jax
import jax.numpy as jnp
from jax.experimental import pallas as pl


def kernel(X, color_table, pos_table):
    raise NotImplementedError("write your pallas kernel here")


# speedup vs baseline: 3.5327x; 3.5327x over previous
import functools

import jax
import jax.numpy as jnp
from jax import lax
from jax.experimental import pallas as pl
from jax.experimental.pallas import tpu as pltpu
from jax.experimental.pallas import tpu_sc as plsc

BATCH = 16384
SEQ = 35
N_BEAKERS = SEQ // 5
IDS_PER_ROW = 4 * N_BEAKERS
D = 64
VOCAB = 7

PAIRS_PER_ROW = IDS_PER_ROW // 2
DP = 2 * D

NUM_CORES = 2
NUM_SUBCORES = 16
NW = NUM_CORES * NUM_SUBCORES

IDX_MINOR = 128
ROWS_TOTAL = BATCH * PAIRS_PER_ROW
IDX_ROWS = ROWS_TOTAL // IDX_MINOR
IDX_ROWS_PER_W = IDX_ROWS // NW
CHUNK_IDX_ROWS = 2
CHUNK_ROWS = CHUNK_IDX_ROWS * IDX_MINOR
N_CHUNKS = IDX_ROWS_PER_W // CHUNK_IDX_ROWS


def _make_sc_gather():
    mesh = plsc.VectorSubcoreMesh(core_axis_name="c", subcore_axis_name="s")

    @functools.partial(
        pl.kernel,
        mesh=mesh,
        out_type=jax.ShapeDtypeStruct((ROWS_TOTAL, DP), jnp.float32),
        scratch_types=[
            pltpu.VMEM((IDX_ROWS_PER_W, IDX_MINOR), jnp.int32),
            pltpu.VMEM((CHUNK_ROWS, DP), jnp.float32),
            pltpu.SemaphoreType.DMA,
        ],
    )
    def sc_gather(table_hbm, idx_hbm, out_hbm, idx_v, rows_v, sem):
        wid = lax.axis_index("s") * NUM_CORES + lax.axis_index("c")
        idx_base = wid * IDX_ROWS_PER_W
        pltpu.sync_copy(idx_hbm.at[pl.ds(idx_base, IDX_ROWS_PER_W)], idx_v)

        def chunk_body(ci, carry):
            copies = []
            for j in range(CHUNK_IDX_ROWS):
                copies.append(pltpu.async_copy(
                    table_hbm.at[idx_v.at[ci * CHUNK_IDX_ROWS + j]],
                    rows_v.at[pl.ds(j * IDX_MINOR, IDX_MINOR)],
                    sem,
                ))
            for c in copies:
                c.wait()
            out_row = (idx_base + ci * CHUNK_IDX_ROWS) * IDX_MINOR
            pltpu.sync_copy(rows_v, out_hbm.at[pl.ds(out_row, CHUNK_ROWS)])
            return carry

        lax.fori_loop(0, N_CHUNKS, chunk_body, 0)

    return sc_gather


_sc_gather = _make_sc_gather()


def kernel(X, color_table, pos_table):
    del pos_table
    ids = X.reshape(BATCH, N_BEAKERS, 5)[:, :, 1:5].astype(jnp.int32)
    ids = ids.reshape(BATCH, PAIRS_PER_ROW, 2)
    pair_ids = ids[:, :, 0] * VOCAB + ids[:, :, 1]
    idx = pair_ids.reshape(IDX_ROWS, IDX_MINOR)
    ptable = jnp.concatenate(
        [jnp.repeat(color_table, VOCAB, axis=0),
         jnp.tile(color_table, (VOCAB, 1))], axis=1)
    rows = _sc_gather(ptable, idx)
    return rows.reshape(BATCH, IDS_PER_ROW * D)


# speedup vs baseline: 9.2793x; 2.6267x over previous
import functools

import jax
import jax.numpy as jnp
from jax import lax
from jax.experimental import pallas as pl
from jax.experimental.pallas import tpu as pltpu
from jax.experimental.pallas import tpu_sc as plsc

BATCH = 16384
SEQ = 35
N_BEAKERS = SEQ // 5
IDS_PER_ROW = 4 * N_BEAKERS
D = 64
VOCAB = 7

PAIRS_PER_ROW = IDS_PER_ROW // 2
DP = 2 * D
PVOCAB = VOCAB * VOCAB

NUM_CORES = 2
NUM_SUBCORES = 16
NW = NUM_CORES * NUM_SUBCORES

IDX_MINOR = 128
ROWS_TOTAL = BATCH * PAIRS_PER_ROW
IDX_ROWS = ROWS_TOTAL // IDX_MINOR
IDX_ROWS_PER_W = IDX_ROWS // NW
CHUNK_IDX_ROWS = 2
CHUNK_ROWS = CHUNK_IDX_ROWS * IDX_MINOR
N_CHUNKS = IDX_ROWS_PER_W // CHUNK_IDX_ROWS


def _make_sc_gather():
    mesh = plsc.VectorSubcoreMesh(core_axis_name="c", subcore_axis_name="s")

    @functools.partial(
        pl.kernel,
        mesh=mesh,
        out_type=jax.ShapeDtypeStruct((ROWS_TOTAL, DP), jnp.float32),
        scratch_types=[
            pltpu.VMEM_SHARED((PVOCAB, DP), jnp.float32),
            pltpu.VMEM((IDX_ROWS_PER_W, IDX_MINOR), jnp.int32),
            pltpu.VMEM((2, CHUNK_ROWS, DP), jnp.float32),
            pltpu.SemaphoreType.DMA,
            pltpu.SemaphoreType.DMA,
        ],
    )
    def sc_gather(table_hbm, idx_hbm, out_hbm, table_v, idx_v, rows_v, sem_g, sem_o):
        wid = lax.axis_index("s") * NUM_CORES + lax.axis_index("c")
        idx_base = wid * IDX_ROWS_PER_W
        @pl.when(lax.axis_index("s") == 0)
        def _():
            pltpu.sync_copy(table_hbm, table_v)

        pltpu.sync_copy(idx_hbm.at[pl.ds(idx_base, IDX_ROWS_PER_W)], idx_v)
        plsc.subcore_barrier()

        def gather_descs(ci, buf):
            return [
                pltpu.make_async_copy(
                    table_v.at[idx_v.at[ci * CHUNK_IDX_ROWS + j]],
                    buf.at[pl.ds(j * IDX_MINOR, IDX_MINOR)],
                    sem_g,
                )
                for j in range(CHUNK_IDX_ROWS)
            ]

        def out_desc(ci, buf):
            out_row = (idx_base + ci * CHUNK_IDX_ROWS) * IDX_MINOR
            return pltpu.make_async_copy(
                buf, out_hbm.at[pl.ds(out_row, CHUNK_ROWS)], sem_o)

        for d in gather_descs(0, rows_v.at[0]):
            d.start()

        def chunk_body(ci, carry):
            buf = rows_v.at[ci % 2]
            nbuf = rows_v.at[(ci + 1) % 2]

            @pl.when(ci >= 1)
            def _():
                out_desc(ci - 1, nbuf).wait()

            @pl.when(ci + 1 < N_CHUNKS)
            def _():
                for d in gather_descs(ci + 1, nbuf):
                    d.start()

            for d in gather_descs(ci, buf):
                d.wait()
            out_desc(ci, buf).start()
            return carry

        lax.fori_loop(0, N_CHUNKS, chunk_body, 0)
        out_desc(N_CHUNKS - 1, rows_v.at[(N_CHUNKS - 1) % 2]).wait()

    return sc_gather


_sc_gather = _make_sc_gather()


def kernel(X, color_table, pos_table):
    del pos_table
    ids = X.reshape(BATCH, N_BEAKERS, 5)[:, :, 1:5].astype(jnp.int32)
    ids = ids.reshape(BATCH, PAIRS_PER_ROW, 2)
    pair_ids = ids[:, :, 0] * VOCAB + ids[:, :, 1]
    idx = pair_ids.reshape(IDX_ROWS, IDX_MINOR)
    ptable = jnp.concatenate(
        [jnp.repeat(color_table, VOCAB, axis=0),
         jnp.tile(color_table, (VOCAB, 1))], axis=1)
    rows = _sc_gather(ptable, idx)
    return rows.reshape(BATCH, IDS_PER_ROW * D)
